# baseline (speedup 1.0000x reference)
import functools

import jax
import jax.numpy as jnp
from jax import lax
from jax.experimental import pallas as pl
from jax.experimental.pallas import tpu as pltpu
from jax.experimental.pallas import tpu_sc as plsc

_DIM = 768
_E = 64
_K = 8
_N = 32768
_T = 256
_G = _N // _T
_FLAT = _N * _K
_NW = 32
_CHUNK = _FLAT // _NW
_LANES = 16


def _router_body(x_ref, wt_ref, w_out, e_out, r_out, cnt_out, base_out, run_ref):
    g = pl.program_id(0)

    @pl.when(g == 0)
    def _():
        run_ref[...] = jnp.zeros_like(run_ref)

    logits = jnp.dot(x_ref[...], wt_ref[...], preferred_element_type=jnp.float32)
    m = jnp.max(logits, axis=1, keepdims=True)
    p = jnp.exp(logits - m)
    probs = p / jnp.sum(p, axis=1, keepdims=True)

    lane = lax.broadcasted_iota(jnp.int32, (_T, _E), 1)
    masks, vals, idxs = [], [], []
    sel = jnp.zeros((_T, _E), jnp.float32)
    cur = probs
    for _ in range(_K):
        mx = jnp.max(cur, axis=1, keepdims=True)
        eq = cur == mx
        first_idx = jnp.min(jnp.where(eq, lane, _E), axis=1, keepdims=True)
        mask = lane == first_idx
        masks.append(mask)
        vals.append(mx)
        idxs.append(first_idx)
        sel = sel + mask.astype(jnp.float32)
        cur = jnp.where(mask, -1.0, cur)

    rows = lax.broadcasted_iota(jnp.int32, (_T, _T), 0)
    cols = lax.broadcasted_iota(jnp.int32, (_T, _T), 1)
    tril = (rows > cols).astype(jnp.float32)
    prefix = jnp.dot(tril, sel, preferred_element_type=jnp.float32)
    rankmat = run_ref[...] + prefix.astype(jnp.int32)

    ranks = [jnp.sum(jnp.where(mk, rankmat, 0), axis=1, keepdims=True)
             for mk in masks]

    w_out[...] = jnp.concatenate(vals, axis=1)
    e_out[...] = jnp.concatenate(idxs, axis=1)
    r_out[...] = jnp.concatenate(ranks, axis=1)

    counts_tile = jnp.sum(sel, axis=0, keepdims=True).astype(jnp.int32)
    new_run = run_ref[...] + counts_tile
    run_ref[...] = new_run

    @pl.when(g == _G - 1)
    def _():
        cnt_out[...] = new_run
        z1 = jnp.zeros((1, 1), jnp.int32)
        b = jnp.concatenate([z1, new_run[:, :-1]], axis=1)
        for sh in (1, 2, 4, 8, 16, 32):
            zs = jnp.zeros((1, sh), jnp.int32)
            b = b + jnp.concatenate([zs, b[:, :-sh]], axis=1)
        base_out[...] = b


_router_call = pl.pallas_call(
    _router_body,
    grid=(_G,),
    in_specs=[
        pl.BlockSpec((_T, _DIM), lambda g: (g, 0)),
        pl.BlockSpec((_DIM, _E), lambda g: (0, 0)),
    ],
    out_specs=[
        pl.BlockSpec((_T, _K), lambda g: (g, 0)),
        pl.BlockSpec((_T, _K), lambda g: (g, 0)),
        pl.BlockSpec((_T, _K), lambda g: (g, 0)),
        pl.BlockSpec((1, _E), lambda g: (0, 0)),
        pl.BlockSpec((1, _E), lambda g: (0, 0)),
    ],
    out_shape=[
        jax.ShapeDtypeStruct((_N, _K), jnp.float32),
        jax.ShapeDtypeStruct((_N, _K), jnp.int32),
        jax.ShapeDtypeStruct((_N, _K), jnp.int32),
        jax.ShapeDtypeStruct((1, _E), jnp.int32),
        jax.ShapeDtypeStruct((1, _E), jnp.int32),
    ],
    scratch_shapes=[pltpu.VMEM((1, _E), jnp.int32)],
    compiler_params=pltpu.CompilerParams(
        dimension_semantics=("arbitrary",)),
)


def _finalize_body(e_ref, r_ref, base_ref, s_out):
    lane = lax.broadcasted_iota(jnp.int32, (_T, _E), 1)
    base_row = base_ref[...]
    cols = []
    for k in range(_K):
        e_k = e_ref[:, k:k + 1]
        onehot = lane == e_k
        b_k = jnp.sum(jnp.where(onehot, base_row, 0), axis=1, keepdims=True)
        cols.append(r_ref[:, k:k + 1] + b_k)
    s_out[...] = jnp.concatenate(cols, axis=1)


_finalize_call = pl.pallas_call(
    _finalize_body,
    grid=(_G,),
    in_specs=[
        pl.BlockSpec((_T, _K), lambda g: (g, 0)),
        pl.BlockSpec((_T, _K), lambda g: (g, 0)),
        pl.BlockSpec((1, _E), lambda g: (0, 0)),
    ],
    out_specs=pl.BlockSpec((_T, _K), lambda g: (g, 0)),
    out_shape=jax.ShapeDtypeStruct((_N, _K), jnp.int32),
)


@functools.cache
def _permute_call():
    @functools.partial(
        pl.kernel,
        mesh=plsc.VectorSubcoreMesh(core_axis_name="c", subcore_axis_name="s",
                                    num_cores=2, num_subcores=16),
        out_type=jax.ShapeDtypeStruct((_FLAT,), jnp.int32),
        scratch_types=[
            pltpu.VMEM((_CHUNK,), jnp.int32),
            pltpu.VMEM((_CHUNK,), jnp.int32),
            pltpu.SemaphoreType.DMA,
        ],
    )
    def permute(s_hbm, iota_hbm, gather_hbm, sv, iv, sem):
        cid = lax.axis_index("c")
        sid = lax.axis_index("s")
        wid = sid * 2 + cid
        start = wid * _CHUNK
        pltpu.sync_copy(s_hbm.at[pl.ds(start, _CHUNK)], sv)
        pltpu.sync_copy(iota_hbm.at[pl.ds(start, _CHUNK)], iv)
        pltpu.async_copy(iv, gather_hbm.at[sv], sem).wait()

    return permute


def kernel(x, W):
    x = x.reshape(-1, _DIM)
    w8, e8, r8, cnt, base = _router_call(x, W.T)
    scatter8 = _finalize_call(e8, r8, base)
    scatter = scatter8.reshape(-1)
    iota = jnp.arange(_FLAT, dtype=jnp.int32)
    gather = _permute_call()(scatter, iota)
    return w8.reshape(-1), gather, scatter, cnt.reshape(-1)


# speedup vs baseline: 1.4668x; 1.4668x over previous
import functools

import jax
import jax.numpy as jnp
from jax import lax
from jax.experimental import pallas as pl
from jax.experimental.pallas import tpu as pltpu
from jax.experimental.pallas import tpu_sc as plsc

_DIM = 768
_E = 64
_K = 8
_N = 32768
_T = 256
_G = _N // _T
_FLAT = _N * _K
_NW = 32
_CHUNK = _FLAT // _NW
_LANES = 16


def _router_body(x_ref, wt_ref, w_out, e_out, r_out, cnt_out, base_out, run_ref):
    g = pl.program_id(0)

    @pl.when(g == 0)
    def _():
        run_ref[...] = jnp.zeros_like(run_ref)

    logits = jnp.dot(x_ref[...], wt_ref[...], preferred_element_type=jnp.float32)
    m = jnp.max(logits, axis=1, keepdims=True)
    p = jnp.exp(logits - m)
    probs = p / jnp.sum(p, axis=1, keepdims=True)

    lane = lax.broadcasted_iota(jnp.int32, (_T, _E), 1)
    masks, vals, idxs = [], [], []
    sel = jnp.zeros((_T, _E), jnp.float32)
    cur = probs
    for _ in range(_K):
        mx = jnp.max(cur, axis=1, keepdims=True)
        eq = cur == mx
        first_idx = jnp.min(jnp.where(eq, lane, _E), axis=1, keepdims=True)
        mask = lane == first_idx
        masks.append(mask)
        vals.append(mx)
        idxs.append(first_idx)
        sel = sel + mask.astype(jnp.float32)
        cur = jnp.where(mask, -1.0, cur)

    rows = lax.broadcasted_iota(jnp.int32, (_T, _T), 0)
    cols = lax.broadcasted_iota(jnp.int32, (_T, _T), 1)
    tril = (rows > cols).astype(jnp.float32)
    prefix = jnp.dot(tril, sel, preferred_element_type=jnp.float32)
    rankmat = run_ref[...] + prefix.astype(jnp.int32)

    ranks = [jnp.sum(jnp.where(mk, rankmat, 0), axis=1, keepdims=True)
             for mk in masks]

    w_out[...] = jnp.concatenate(vals, axis=1)
    e_out[...] = jnp.concatenate(idxs, axis=1)
    r_out[...] = jnp.concatenate(ranks, axis=1)

    counts_tile = jnp.sum(sel, axis=0, keepdims=True).astype(jnp.int32)
    new_run = run_ref[...] + counts_tile
    run_ref[...] = new_run

    @pl.when(g == _G - 1)
    def _():
        cnt_out[...] = new_run
        z1 = jnp.zeros((1, 1), jnp.int32)
        b = jnp.concatenate([z1, new_run[:, :-1]], axis=1)
        for sh in (1, 2, 4, 8, 16, 32):
            zs = jnp.zeros((1, sh), jnp.int32)
            b = b + jnp.concatenate([zs, b[:, :-sh]], axis=1)
        base_out[...] = b


_router_call = pl.pallas_call(
    _router_body,
    grid=(_G,),
    in_specs=[
        pl.BlockSpec((_T, _DIM), lambda g: (g, 0)),
        pl.BlockSpec((_DIM, _E), lambda g: (0, 0)),
    ],
    out_specs=[
        pl.BlockSpec((_T, _K), lambda g: (g, 0)),
        pl.BlockSpec((_T, _K), lambda g: (g, 0)),
        pl.BlockSpec((_T, _K), lambda g: (g, 0)),
        pl.BlockSpec((1, _E), lambda g: (0, 0)),
        pl.BlockSpec((1, _E), lambda g: (0, 0)),
    ],
    out_shape=[
        jax.ShapeDtypeStruct((_N, _K), jnp.float32),
        jax.ShapeDtypeStruct((_N, _K), jnp.int32),
        jax.ShapeDtypeStruct((_N, _K), jnp.int32),
        jax.ShapeDtypeStruct((1, _E), jnp.int32),
        jax.ShapeDtypeStruct((1, _E), jnp.int32),
    ],
    scratch_shapes=[pltpu.VMEM((1, _E), jnp.int32)],
    compiler_params=pltpu.CompilerParams(
        dimension_semantics=("arbitrary",)),
)


def _finalize_body(e_ref, r_ref, base_ref, s_out):
    lane = lax.broadcasted_iota(jnp.int32, (_T, _E), 1)
    base_row = base_ref[...]
    cols = []
    for k in range(_K):
        e_k = e_ref[:, k:k + 1]
        onehot = lane == e_k
        b_k = jnp.sum(jnp.where(onehot, base_row, 0), axis=1, keepdims=True)
        cols.append(r_ref[:, k:k + 1] + b_k)
    s_out[...] = jnp.concatenate(cols, axis=1)


_finalize_call = pl.pallas_call(
    _finalize_body,
    grid=(_G,),
    in_specs=[
        pl.BlockSpec((_T, _K), lambda g: (g, 0)),
        pl.BlockSpec((_T, _K), lambda g: (g, 0)),
        pl.BlockSpec((1, _E), lambda g: (0, 0)),
    ],
    out_specs=pl.BlockSpec((_T, _K), lambda g: (g, 0)),
    out_shape=jax.ShapeDtypeStruct((_N, _K), jnp.int32),
)


@functools.cache
def _permute_call():
    sub = _FLAT // 16
    wb = _FLAT // 32

    @functools.partial(
        pl.kernel,
        mesh=plsc.VectorSubcoreMesh(core_axis_name="c", subcore_axis_name="s",
                                    num_cores=2, num_subcores=16),
        out_type=jax.ShapeDtypeStruct((_FLAT,), jnp.int32),
        scratch_types=[
            pltpu.VMEM((sub,), jnp.int32),
            pltpu.VMEM((sub,), jnp.int32),
            pltpu.VMEM_SHARED((_FLAT,), jnp.int32),
        ],
    )
    def permute(s_hbm, iota_hbm, gather_hbm, sv, iv, buf):
        cid = lax.axis_index("c")
        sid = lax.axis_index("s")
        start = sid * sub
        pltpu.sync_copy(s_hbm.at[pl.ds(start, sub)], sv)
        pltpu.sync_copy(iota_hbm.at[pl.ds(start, sub)], iv)
        pltpu.sync_copy(iv, buf.at[sv])
        plsc.subcore_barrier()
        off = cid * (_FLAT // 2) + sid * wb
        pltpu.sync_copy(buf.at[pl.ds(off, wb)], gather_hbm.at[pl.ds(off, wb)])

    return permute


def kernel(x, W):
    x = x.reshape(-1, _DIM)
    w8, e8, r8, cnt, base = _router_call(x, W.T)
    scatter8 = _finalize_call(e8, r8, base)
    scatter = scatter8.reshape(-1)
    iota = jnp.arange(_FLAT, dtype=jnp.int32)
    gather = _permute_call()(scatter, iota)
    return w8.reshape(-1), gather, scatter, cnt.reshape(-1)


# speedup vs baseline: 2.1678x; 1.4779x over previous
import functools

import jax
import jax.numpy as jnp
from jax import lax
from jax.experimental import pallas as pl
from jax.experimental.pallas import tpu as pltpu
from jax.experimental.pallas import tpu_sc as plsc

_DIM = 768
_E = 64
_K = 8
_N = 32768
_T = 256
_G = _N // _T
_FLAT = _N * _K
_NW = 32
_CHUNK = _FLAT // _NW
_LANES = 16


def _router_body(x_ref, wt_ref, w_out, e_out, r_out, cnt_out, base_out, run_ref):
    g = pl.program_id(0)

    @pl.when(g == 0)
    def _():
        run_ref[...] = jnp.zeros_like(run_ref)

    logits = jnp.dot(x_ref[...], wt_ref[...], preferred_element_type=jnp.float32)
    m = jnp.max(logits, axis=1, keepdims=True)
    p = jnp.exp(logits - m)
    denom = jnp.sum(p, axis=1, keepdims=True)

    masks, vals = [], []
    sel = jnp.zeros((_T, _E), jnp.float32)
    cur = p
    for _ in range(_K):
        mx = jnp.max(cur, axis=1, keepdims=True)
        mask = cur == mx
        sel = sel + mask.astype(jnp.float32)
        cur = jnp.where(mask, -1.0, cur)
        masks.append(mask)
        vals.append(mx)

    rows = lax.broadcasted_iota(jnp.int32, (_T, _T), 0)
    cols = lax.broadcasted_iota(jnp.int32, (_T, _T), 1)
    tril = (rows > cols).astype(jnp.float32)
    prefix = jnp.dot(tril, sel, preferred_element_type=jnp.float32)
    rankmat = run_ref[...] + prefix

    lane = lax.broadcasted_iota(jnp.int32, (_T, _E), 1)
    payload = rankmat * 64.0 + lane.astype(jnp.float32)
    pays = [jnp.sum(jnp.where(mk, payload, 0.0), axis=1, keepdims=True)
            for mk in masks]
    pay8 = jnp.concatenate(pays, axis=1).astype(jnp.int32)

    w_out[...] = jnp.concatenate(vals, axis=1) / denom
    e_out[...] = pay8 & 63
    r_out[...] = pay8 >> 6

    counts_tile = jnp.sum(sel, axis=0, keepdims=True)
    new_run = run_ref[...] + counts_tile
    run_ref[...] = new_run

    @pl.when(g == _G - 1)
    def _():
        cnt = new_run.astype(jnp.int32)
        cnt_out[...] = cnt
        z1 = jnp.zeros((1, 1), jnp.int32)
        b = jnp.concatenate([z1, cnt[:, :-1]], axis=1)
        for sh in (1, 2, 4, 8, 16, 32):
            zs = jnp.zeros((1, sh), jnp.int32)
            b = b + jnp.concatenate([zs, b[:, :-sh]], axis=1)
        base_out[...] = b


_router_call = pl.pallas_call(
    _router_body,
    grid=(_G,),
    in_specs=[
        pl.BlockSpec((_T, _DIM), lambda g: (g, 0)),
        pl.BlockSpec((_DIM, _E), lambda g: (0, 0)),
    ],
    out_specs=[
        pl.BlockSpec((_T, _K), lambda g: (g, 0)),
        pl.BlockSpec((_T, _K), lambda g: (g, 0)),
        pl.BlockSpec((_T, _K), lambda g: (g, 0)),
        pl.BlockSpec((1, _E), lambda g: (0, 0)),
        pl.BlockSpec((1, _E), lambda g: (0, 0)),
    ],
    out_shape=[
        jax.ShapeDtypeStruct((_N, _K), jnp.float32),
        jax.ShapeDtypeStruct((_N, _K), jnp.int32),
        jax.ShapeDtypeStruct((_N, _K), jnp.int32),
        jax.ShapeDtypeStruct((1, _E), jnp.int32),
        jax.ShapeDtypeStruct((1, _E), jnp.int32),
    ],
    scratch_shapes=[pltpu.VMEM((1, _E), jnp.float32)],
    compiler_params=pltpu.CompilerParams(
        dimension_semantics=("arbitrary",)),
)


def _finalize_body(e_ref, r_ref, base_ref, s_out):
    lane = lax.broadcasted_iota(jnp.int32, (_T, _E), 1)
    base_row = base_ref[...]
    cols = []
    for k in range(_K):
        e_k = e_ref[:, k:k + 1]
        onehot = lane == e_k
        b_k = jnp.sum(jnp.where(onehot, base_row, 0), axis=1, keepdims=True)
        cols.append(r_ref[:, k:k + 1] + b_k)
    s_out[...] = jnp.concatenate(cols, axis=1)


_finalize_call = pl.pallas_call(
    _finalize_body,
    grid=(_G,),
    in_specs=[
        pl.BlockSpec((_T, _K), lambda g: (g, 0)),
        pl.BlockSpec((_T, _K), lambda g: (g, 0)),
        pl.BlockSpec((1, _E), lambda g: (0, 0)),
    ],
    out_specs=pl.BlockSpec((_T, _K), lambda g: (g, 0)),
    out_shape=jax.ShapeDtypeStruct((_N, _K), jnp.int32),
)


@functools.cache
def _permute_call():
    sub = _FLAT // 16
    wb = _FLAT // 32

    @functools.partial(
        pl.kernel,
        mesh=plsc.VectorSubcoreMesh(core_axis_name="c", subcore_axis_name="s",
                                    num_cores=2, num_subcores=16),
        out_type=jax.ShapeDtypeStruct((_FLAT,), jnp.int32),
        scratch_types=[
            pltpu.VMEM((sub,), jnp.int32),
            pltpu.VMEM((sub,), jnp.int32),
            pltpu.VMEM_SHARED((_FLAT,), jnp.int32),
        ],
    )
    def permute(s_hbm, iota_hbm, gather_hbm, sv, iv, buf):
        cid = lax.axis_index("c")
        sid = lax.axis_index("s")
        start = sid * sub
        pltpu.sync_copy(s_hbm.at[pl.ds(start, sub)], sv)
        pltpu.sync_copy(iota_hbm.at[pl.ds(start, sub)], iv)
        pltpu.sync_copy(iv, buf.at[sv])
        plsc.subcore_barrier()
        off = cid * (_FLAT // 2) + sid * wb
        pltpu.sync_copy(buf.at[pl.ds(off, wb)], gather_hbm.at[pl.ds(off, wb)])

    return permute


def kernel(x, W):
    x = x.reshape(-1, _DIM)
    w8, e8, r8, cnt, base = _router_call(x, W.T)
    scatter8 = _finalize_call(e8, r8, base)
    scatter = scatter8.reshape(-1)
    iota = jnp.arange(_FLAT, dtype=jnp.int32)
    gather = _permute_call()(scatter, iota)
    return w8.reshape(-1), gather, scatter, cnt.reshape(-1)
